# baseline (speedup 1.0000x reference)
import functools

import jax
import jax.numpy as jnp
from jax import lax
from jax.experimental import pallas as pl
from jax.experimental.pallas import tpu as pltpu
from jax.experimental.pallas import tpu_sc as plsc

D_MODEL = 64
SCALE = 8.0
NC, NS = 2, 16
NW = NC * NS
VOCAB = 1000000
PACK = VOCAB // 2
BATCH = 4096
SEQ = 200
LANES = 16
BLK = 128
NBUF = 3


def _iota16():
    return lax.iota(jnp.int32, LANES)


def _lookup_body(xt_hbm, dense_hbm, out_hbm, idx_v, hall,
                 gb0, gb1, gb2, ob0, ob1, sg0, sg1, sg2, so0, so1):
    c = lax.axis_index("c")
    s = lax.axis_index("s")
    wid = s * NC + c
    i0 = wid * BLK
    gbs = (gb0, gb1, gb2)
    obs = (ob0, ob1)
    sgs, sos = (sg0, sg1, sg2), (so0, so1)
    it16 = _iota16()
    kvecs = [it16 + 16 * g for g in range(8)]

    pltpu.sync_copy(xt_hbm.at[:, pl.ds(i0, BLK)], idx_v)

    @plsc.parallel_loop(0, SEQ, unroll=2)
    def _prep(t):
        for g in range(8):
            sl = pl.ds(16 * g, LANES)
            v = idx_v[t, sl]
            hall[t, sl] = (v & 1) * D_MODEL
            idx_v[t, sl] = lax.shift_right_logical(v, 1)

    def issue_gather(t, r):
        pltpu.async_copy(dense_hbm.at[idx_v.at[t]], gbs[r], sgs[r])

    def wait_gather(t, r):
        pltpu.make_async_copy(dense_hbm.at[idx_v.at[t]], gbs[r], sgs[r]).wait()

    def issue_out(t, r):
        for dr in range(8):
            pltpu.async_copy(obs[r].at[dr], out_hbm.at[t, dr, wid], sos[r])

    def wait_out(t, r):
        for dr in range(8):
            pltpu.make_async_copy(obs[r].at[dr], out_hbm.at[t, dr, wid],
                                  sos[r]).wait()

    def transpose_block(t, rg, ro):
        gb, ob = gbs[rg], obs[ro]
        hvs = [hall[t, pl.ds(16 * g, LANES)] for g in range(8)]

        @plsc.parallel_loop(0, 8)
        def rowdr(dr):
            d0 = dr * 8
            for dl in range(8):
                d = d0 + dl
                vs = [plsc.load_gather(gb, [kvecs[g], hvs[g] + d])
                      for g in range(8)]
                for g in range(8):
                    ob[dr, dl, pl.ds(16 * g, LANES)] = vs[g] * SCALE

    for t in range(NBUF):
        issue_gather(t, t)

    def six_body(m, _):
        base = 6 * m
        for j in range(6):
            t = base + j
            rg = j % NBUF
            ro = j % 2
            wait_gather(t, rg)

            @pl.when(t >= 2)
            def _():
                wait_out(t - 2, ro)

            transpose_block(t, rg, ro)
            issue_out(t, ro)

            @pl.when(t + NBUF < SEQ)
            def _():
                issue_gather(t + NBUF, rg)
        return 0

    lax.fori_loop(0, 33, six_body, 0)
    for t in (198, 199):
        rg = t % NBUF
        ro = t % 2
        wait_gather(t, rg)
        wait_out(t - 2, ro)
        transpose_block(t, rg, ro)
        issue_out(t, ro)
    wait_out(198, 0)
    wait_out(199, 1)


_lookup = functools.partial(
    pl.kernel,
    out_type=jax.ShapeDtypeStruct((SEQ, 8, NW, 8, BLK), jnp.float32),
    mesh=plsc.VectorSubcoreMesh(core_axis_name="c", subcore_axis_name="s"),
    compiler_params=pltpu.CompilerParams(use_tc_tiling_on_sc=False,
                                         needs_layout_passes=False),
    scratch_types=(
        [pltpu.VMEM((SEQ, BLK), jnp.int32) for _ in range(2)]
        + [pltpu.VMEM((BLK, BLK), jnp.float32) for _ in range(NBUF)]
        + [pltpu.VMEM((8, 8, BLK), jnp.float32) for _ in range(2)]
        + [pltpu.SemaphoreType.DMA for _ in range(NBUF + 2)]
    ),
)(_lookup_body)


@jax.jit
def kernel(x, table):
    dense = table.reshape(PACK, 2 * D_MODEL)
    out5 = _lookup(x.T, dense)
    out = out5.transpose(0, 1, 3, 2, 4).reshape(SEQ, D_MODEL, BATCH)
    return out.transpose(2, 0, 1)


# speedup vs baseline: 1.2338x; 1.2338x over previous
import functools

import jax
import jax.numpy as jnp
from jax import lax
from jax.experimental import pallas as pl
from jax.experimental.pallas import tpu as pltpu
from jax.experimental.pallas import tpu_sc as plsc

D_MODEL = 64
SCALE = 8.0
NC, NS = 2, 16
NW = NC * NS
XROWS = 4096
SEQ = 200
RPW = XROWS // NW
SPLIT = 128
LANES = 16
NBUF = 4
AHEAD = 2


def _body(x_hbm, table_hbm, out_hbm, idx_v, *rest):
    bufs = rest[:NBUF]
    sgs = rest[NBUF:2 * NBUF]
    sos = rest[2 * NBUF:3 * NBUF]
    c = lax.axis_index("c")
    s = lax.axis_index("s")
    wid = s * NC + c
    base = wid * RPW
    pltpu.sync_copy(x_hbm.at[pl.ds(base, RPW)], idx_v)

    def gather_parts(r, b):
        return (
            (table_hbm.at[idx_v.at[r, pl.ds(0, SPLIT)]],
             bufs[b].at[pl.ds(0, SPLIT)]),
            (table_hbm.at[idx_v.at[r, pl.ds(SPLIT, SEQ - SPLIT)]],
             bufs[b].at[pl.ds(SPLIT, SEQ - SPLIT)]),
        )

    def issue_gather(r, b):
        for src, dst in gather_parts(r, b):
            pltpu.async_copy(src, dst, sgs[b])

    def wait_gather(r, b):
        for src, dst in gather_parts(r, b):
            pltpu.make_async_copy(src, dst, sgs[b]).wait()

    def issue_out(r, b):
        pltpu.async_copy(bufs[b], out_hbm.at[base + r], sos[b])

    def wait_out(r, b):
        pltpu.make_async_copy(bufs[b], out_hbm.at[base + r], sos[b]).wait()

    def scale(b):
        buf = bufs[b]

        @plsc.parallel_loop(0, SEQ // 4, unroll=2)
        def row4(i):
            q = i * 4
            for v in range(4):
                for u in range(D_MODEL // LANES):
                    sl = pl.ds(u * LANES, LANES)
                    buf[q + v, sl] = buf[q + v, sl] * SCALE

    def step(r, b, first=False):
        wait_gather(r, b)
        scale(b)
        issue_out(r, b)
        rn = r + AHEAD
        bn = (b + AHEAD) % NBUF
        if not first:
            wait_out(rn - NBUF, bn)
        issue_gather(rn, bn)

    for r in range(AHEAD):
        issue_gather(r, r)
    for b in range(NBUF):
        step(b, b, first=(b < AHEAD))

    def block(g, _):
        r0 = g * NBUF
        for b in range(NBUF):
            step(r0 + b, b)
        return 0

    lax.fori_loop(1, RPW // NBUF - 1, block, 0)

    r0 = RPW - NBUF
    for b in range(AHEAD):
        step(r0 + b, b)
    for b in range(AHEAD, NBUF):
        r = r0 + b
        wait_gather(r, b)
        scale(b)
        issue_out(r, b)
    for b in range(NBUF):
        wait_out(r0 + b, b)


_sc_call = functools.partial(
    pl.kernel,
    out_type=jax.ShapeDtypeStruct((XROWS, SEQ, D_MODEL), jnp.float32),
    mesh=plsc.VectorSubcoreMesh(core_axis_name="c", subcore_axis_name="s"),
    compiler_params=pltpu.CompilerParams(use_tc_tiling_on_sc=False),
    scratch_types=(
        [pltpu.VMEM((RPW, SEQ), jnp.int32)]
        + [pltpu.VMEM((SEQ, D_MODEL), jnp.float32) for _ in range(NBUF)]
        + [pltpu.SemaphoreType.DMA for _ in range(2 * NBUF)]
    ),
)(_body)


@jax.jit
def kernel(x, table):
    return _sc_call(x, table)
